# baseline (speedup 1.0000x reference)
import functools

import jax
import jax.numpy as jnp
from jax import lax
from jax.experimental import pallas as pl
from jax.experimental.pallas import tpu as pltpu
from jax.experimental.pallas import tpu_sc as plsc

_B, _S, _D = 4, 8192, 1024
_NC, _NS = 2, 16
_NW = _NC * _NS
_SROWS = _S // _NW
_K = 32
_CH = _K * _D
_NCHUNK = _SROWS // _K
_NVEC = _CH // 16


def _sc_body(x_hbm, e_hbm, o_hbm, x_buf, e_buf):
    cid = lax.axis_index("c")
    sid = lax.axis_index("s")
    wid = sid * _NC + cid
    e_base = wid * _SROWS * _D

    @pl.loop(0, _NCHUNK)
    def _chunk(ci):
        eo = e_base + ci * _CH
        pltpu.sync_copy(e_hbm.at[pl.ds(eo, _CH)], e_buf)
        for b in range(_B):
            xo = b * _S * _D + eo
            pltpu.sync_copy(x_hbm.at[pl.ds(xo, _CH)], x_buf)

            @plsc.parallel_loop(0, _NVEC, unroll=8)
            def _vec(j):
                off = j * 16
                x_buf[pl.ds(off, 16)] = x_buf[pl.ds(off, 16)] + e_buf[pl.ds(off, 16)]

            pltpu.sync_copy(x_buf, o_hbm.at[pl.ds(xo, _CH)])


def kernel(x, emb_weight):
    mesh = plsc.VectorSubcoreMesh(core_axis_name="c", subcore_axis_name="s")
    k = pl.kernel(
        _sc_body,
        out_type=jax.ShapeDtypeStruct((_B * _S * _D,), jnp.float32),
        mesh=mesh,
        scratch_types=[
            pltpu.VMEM((_CH,), jnp.float32),
            pltpu.VMEM((_CH,), jnp.float32),
        ],
    )
    out = k(x.reshape(-1), emb_weight.reshape(-1))
    return out.reshape(x.shape)


# speedup vs baseline: 1.1330x; 1.1330x over previous
import jax
import jax.numpy as jnp
from jax import lax
from jax.experimental import pallas as pl
from jax.experimental.pallas import tpu as pltpu
from jax.experimental.pallas import tpu_sc as plsc

_B, _S, _D = 4, 8192, 1024
_NC, _NS = 2, 16
_NW = _NC * _NS
_ROWS = _B * _S // _NW
_K = 16
_CHE = _K * _D
_NCH = _ROWS // _K
_NVEC = _CHE // 16
_XTOT = _B * _S * _D
_ETOT = _S * _D


def _sc_body(x_hbm, e_hbm, o_hbm,
             xb0, xb1, eb0, eb1, ob0, ob1,
             slx0, slx1, sle0, sle1, sst0, sst1):
    cid = lax.axis_index("c")
    sid = lax.axis_index("s")
    wid = sid * _NC + cid
    xbase = wid * _ROWS * _D
    ebase = (wid % (_NW // _B)) * _ROWS * _D

    xb = (xb0, xb1)
    eb = (eb0, eb1)
    ob = (ob0, ob1)
    slx = (slx0, slx1)
    sle = (sle0, sle1)
    sst = (sst0, sst1)

    def xoff(t):
        return jnp.minimum(xbase + t * _CHE, _XTOT - _CHE)

    def eoff(t):
        return jnp.minimum(ebase + t * _CHE, _ETOT - _CHE)

    def issue_loads(t, p):
        pltpu.async_copy(x_hbm.at[pl.ds(xoff(t), _CHE)], xb[p], slx[p])
        pltpu.async_copy(e_hbm.at[pl.ds(eoff(t), _CHE)], eb[p], sle[p])

    def wait_loads(t, p):
        pltpu.make_async_copy(x_hbm.at[pl.ds(xoff(t), _CHE)], xb[p], slx[p]).wait()
        pltpu.make_async_copy(e_hbm.at[pl.ds(eoff(t), _CHE)], eb[p], sle[p]).wait()

    def compute(p):
        xr, er, orr = xb[p], eb[p], ob[p]

        @plsc.parallel_loop(0, _NVEC, unroll=8)
        def _vec(j):
            off = j * 16
            orr[pl.ds(off, 16)] = xr[pl.ds(off, 16)] + er[pl.ds(off, 16)]

    def issue_store(t, p):
        pltpu.async_copy(ob[p], o_hbm.at[pl.ds(xoff(t), _CHE)], sst[p])

    def wait_store(t, p):
        pltpu.make_async_copy(ob[p], o_hbm.at[pl.ds(xoff(t), _CHE)], sst[p]).wait()

    for p in range(2):
        issue_loads(jnp.int32(p), p)
    for p in range(2):
        t = jnp.int32(p)
        wait_loads(t, p)
        compute(p)
        issue_store(t, p)
        issue_loads(t + 2, p)

    @pl.loop(1, _NCH // 2)
    def _pipe(i):
        t0 = i * 2
        for p in range(2):
            t = t0 + p
            wait_loads(t, p)
            wait_store(t - 2, p)
            compute(p)
            issue_store(t, p)
            issue_loads(t + 2, p)

    for p in range(2):
        t = jnp.int32(_NCH - 2 + p)
        wait_store(t, p)
        wait_loads(t + 2, p)


def kernel(x, emb_weight):
    mesh = plsc.VectorSubcoreMesh(core_axis_name="c", subcore_axis_name="s")
    k = pl.kernel(
        _sc_body,
        out_type=jax.ShapeDtypeStruct((_XTOT,), jnp.float32),
        mesh=mesh,
        scratch_types=[
            pltpu.VMEM((_CHE,), jnp.float32),
            pltpu.VMEM((_CHE,), jnp.float32),
            pltpu.VMEM((_CHE,), jnp.float32),
            pltpu.VMEM((_CHE,), jnp.float32),
            pltpu.VMEM((_CHE,), jnp.float32),
            pltpu.VMEM((_CHE,), jnp.float32),
            pltpu.SemaphoreType.DMA,
            pltpu.SemaphoreType.DMA,
            pltpu.SemaphoreType.DMA,
            pltpu.SemaphoreType.DMA,
            pltpu.SemaphoreType.DMA,
            pltpu.SemaphoreType.DMA,
        ],
    )
    out = k(x.reshape(-1), emb_weight.reshape(-1))
    return out.reshape(x.shape)


# speedup vs baseline: 3.5800x; 3.1598x over previous
import jax
import jax.numpy as jnp
from jax import lax
from jax.experimental import pallas as pl
from jax.experimental.pallas import tpu as pltpu
from jax.experimental.pallas import tpu_sc as plsc

_B, _S, _D = 4, 8192, 1024
_NC, _NS = 2, 16
_NW = _NC * _NS
_SROWS = _S // _NW
_K = 16
_NCH = _SROWS // _K
_NVEC = _K * _D // 16
_XROWS = _B * _S
_EROWS = _S


def _sc_body(x_hbm, e_hbm, o_hbm,
             xb0, xb1, eb0, eb1, ob0, ob1,
             slx0, slx1, sle0, sle1, sst0, sst1):
    cid = lax.axis_index("c")
    sid = lax.axis_index("s")
    wid = sid * _NC + cid
    sbase = wid * _SROWS

    xb = (xb0, xb1)
    eb = (eb0, eb1)
    ob = (ob0, ob1)
    slx = (slx0, slx1)
    sle = (sle0, sle1)
    sst = (sst0, sst1)

    def xrow(ci, b):
        return jnp.minimum(b * _S + sbase + ci * _K, _XROWS - _K)

    def erow(ci):
        return jnp.minimum(sbase + ci * _K, _EROWS - _K)

    def issue_xload(ci, b, p):
        pltpu.async_copy(x_hbm.at[pl.ds(xrow(ci, b), _K)], xb[p], slx[p])

    def wait_xload(ci, b, p):
        pltpu.make_async_copy(x_hbm.at[pl.ds(xrow(ci, b), _K)], xb[p], slx[p]).wait()

    def issue_eload(ci, pe):
        pltpu.async_copy(e_hbm.at[pl.ds(erow(ci), _K)], eb[pe], sle[pe])

    def wait_eload(ci, pe):
        pltpu.make_async_copy(e_hbm.at[pl.ds(erow(ci), _K)], eb[pe], sle[pe]).wait()

    def compute(p, pe):
        xr, er, orr = xb[p], eb[pe], ob[p]

        @plsc.parallel_loop(0, _NVEC, unroll=8)
        def _vec(j):
            i = j >> 6
            c = pl.multiple_of((j & 63) << 4, 16)
            orr[i, pl.ds(c, 16)] = xr[i, pl.ds(c, 16)] + er[i, pl.ds(c, 16)]

    def issue_store(ci, b, p):
        pltpu.async_copy(ob[p], o_hbm.at[pl.ds(xrow(ci, b), _K)], sst[p])

    def wait_store(ci, b, p):
        pltpu.make_async_copy(ob[p], o_hbm.at[pl.ds(xrow(ci, b), _K)], sst[p]).wait()

    def chunk_pair(i0, first):
        for dc in range(2):
            ci = i0 * 2 + dc
            for b in range(_B):
                p = b & 1
                wait_xload(ci, b, p)
                if b == 0:
                    wait_eload(ci, dc)
                if first and dc == 0 and b < 2:
                    pass
                elif b >= 2:
                    wait_store(ci, b - 2, p)
                else:
                    wait_store(ci - 1, b + 2, p)
                compute(p, dc)
                issue_store(ci, b, p)
                t2 = ci * _B + b + 2
                issue_xload(t2 // _B, t2 % _B, p)
            issue_eload(ci + 2, dc)

    issue_eload(0, 0)
    issue_eload(1, 1)
    issue_xload(0, 0, 0)
    issue_xload(0, 1, 1)
    chunk_pair(0, True)

    @pl.loop(1, _NCH // 2)
    def _pipe(i0):
        chunk_pair(i0, False)

    for b2 in range(2):
        p = b2 & 1
        wait_store(_NCH - 1, 2 + b2, p)
        nb = (_NCH - 1) * _B + 2 + b2 + 2
        wait_xload(nb // _B, nb % _B, p)
    wait_eload(_NCH, 0)
    wait_eload(_NCH + 1, 1)


def kernel(x, emb_weight):
    mesh = plsc.VectorSubcoreMesh(core_axis_name="c", subcore_axis_name="s")
    k = pl.kernel(
        _sc_body,
        out_type=jax.ShapeDtypeStruct((_XROWS, _D), jnp.float32),
        mesh=mesh,
        compiler_params=pltpu.CompilerParams(use_tc_tiling_on_sc=True),
        scratch_types=[
            pltpu.VMEM((_K, _D), jnp.float32),
            pltpu.VMEM((_K, _D), jnp.float32),
            pltpu.VMEM((_K, _D), jnp.float32),
            pltpu.VMEM((_K, _D), jnp.float32),
            pltpu.VMEM((_K, _D), jnp.float32),
            pltpu.VMEM((_K, _D), jnp.float32),
            pltpu.SemaphoreType.DMA,
            pltpu.SemaphoreType.DMA,
            pltpu.SemaphoreType.DMA,
            pltpu.SemaphoreType.DMA,
            pltpu.SemaphoreType.DMA,
            pltpu.SemaphoreType.DMA,
        ],
    )
    out = k(x.reshape(_XROWS, _D), emb_weight)
    return out.reshape(x.shape)


# speedup vs baseline: 3.8170x; 1.0662x over previous
import jax
import jax.numpy as jnp
from jax import lax
from jax.experimental import pallas as pl
from jax.experimental.pallas import tpu as pltpu
from jax.experimental.pallas import tpu_sc as plsc

_B, _S, _D = 4, 8192, 1024
_NC, _NS = 2, 16
_NW = _NC * _NS
_SROWS = _S // _NW
_K = 8
_NCH = _SROWS // _K
_NVEC = _K * _D // 16
_XROWS = _B * _S
_EROWS = _S


def _sc_body(x_hbm, e_hbm, o_hbm,
             xb0, xb1, xb2, xb3, ob0, ob1, ob2, ob3, eb0, eb1,
             slx0, slx1, slx2, slx3, sst0, sst1, sst2, sst3, sle0, sle1):
    cid = lax.axis_index("c")
    sid = lax.axis_index("s")
    wid = sid * _NC + cid
    sbase = wid * _SROWS

    xb = (xb0, xb1, xb2, xb3)
    ob = (ob0, ob1, ob2, ob3)
    eb = (eb0, eb1)
    slx = (slx0, slx1, slx2, slx3)
    sst = (sst0, sst1, sst2, sst3)
    sle = (sle0, sle1)

    def xrow(ci, b):
        return jnp.minimum(b * _S + sbase + ci * _K, _XROWS - _K)

    def erow(ci):
        return jnp.minimum(sbase + ci * _K, _EROWS - _K)

    def issue_xload(ci, b):
        pltpu.async_copy(x_hbm.at[pl.ds(xrow(ci, b), _K)], xb[b], slx[b])

    def wait_xload(ci, b):
        pltpu.make_async_copy(x_hbm.at[pl.ds(xrow(ci, b), _K)], xb[b], slx[b]).wait()

    def issue_eload(ci, pe):
        pltpu.async_copy(e_hbm.at[pl.ds(erow(ci), _K)], eb[pe], sle[pe])

    def wait_eload(ci, pe):
        pltpu.make_async_copy(e_hbm.at[pl.ds(erow(ci), _K)], eb[pe], sle[pe]).wait()

    def compute(b, pe):
        xr, er, orr = xb[b], eb[pe], ob[b]

        @plsc.parallel_loop(0, _NVEC, unroll=8)
        def _vec(j):
            i = j >> 6
            c = pl.multiple_of((j & 63) << 4, 16)
            orr[i, pl.ds(c, 16)] = xr[i, pl.ds(c, 16)] + er[i, pl.ds(c, 16)]

    def issue_store(ci, b):
        pltpu.async_copy(ob[b], o_hbm.at[pl.ds(xrow(ci, b), _K)], sst[b])

    def wait_store(ci, b):
        pltpu.make_async_copy(ob[b], o_hbm.at[pl.ds(xrow(ci, b), _K)], sst[b]).wait()

    def do_chunk(ci, pe, first):
        for b in range(_B):
            wait_xload(ci, b)
            if b == 0:
                wait_eload(ci, pe)
            if not first:
                wait_store(ci - 1, b)
            compute(b, pe)
            issue_store(ci, b)
            issue_xload(ci + 1, b)
        issue_eload(ci + 2, pe)

    issue_eload(0, 0)
    issue_eload(1, 1)
    for b in range(_B):
        issue_xload(0, b)
    do_chunk(0, 0, True)
    do_chunk(1, 1, False)

    @pl.loop(1, _NCH // 2)
    def _pipe(i0):
        do_chunk(i0 * 2, 0, False)
        do_chunk(i0 * 2 + 1, 1, False)

    for b in range(_B):
        wait_store(_NCH - 1, b)
        wait_xload(_NCH, b)
    wait_eload(_NCH, 0)
    wait_eload(_NCH + 1, 1)


def kernel(x, emb_weight):
    mesh = plsc.VectorSubcoreMesh(core_axis_name="c", subcore_axis_name="s")
    k = pl.kernel(
        _sc_body,
        out_type=jax.ShapeDtypeStruct((_XROWS, _D), jnp.float32),
        mesh=mesh,
        compiler_params=pltpu.CompilerParams(use_tc_tiling_on_sc=True),
        scratch_types=(
            [pltpu.VMEM((_K, _D), jnp.float32) for _ in range(10)]
            + [pltpu.SemaphoreType.DMA for _ in range(10)]
        ),
    )
    out = k(x.reshape(_XROWS, _D), emb_weight)
    return out.reshape(x.shape)
